# baseline (speedup 1.0000x reference)
import jax
import jax.numpy as jnp
from jax import lax
from jax.experimental import pallas as pl
from jax.experimental.pallas import tpu as pltpu
from jax.experimental.pallas import tpu_sc as plsc

N_NODES = 16384
DIM = 32
NUM_CORES = 2
NUM_SUBCORES = 16
NUM_WORKERS = NUM_CORES * NUM_SUBCORES
B_PER_W = N_NODES // NUM_WORKERS


def _gather_body(ids_hbm, cid_hbm, table_hbm, nodes_out, char_out,
                 idx_v, cidx_v, sem, csem):
    wid = lax.axis_index("s") * NUM_CORES + lax.axis_index("c")
    base = wid * B_PER_W

    pltpu.sync_copy(ids_hbm.at[pl.ds(base, B_PER_W)], idx_v)

    @pl.when(wid == 0)
    def _():
        pltpu.sync_copy(cid_hbm, cidx_v.at[pl.ds(0, 1)])
        cv = cidx_v[...]
        pltpu.async_copy(table_hbm.at[pl.ds(cv[0], 1), :], char_out, csem)

    def issue(g, _):
        v = idx_v[pl.ds(g * 16, 16)]
        for l in range(16):
            pltpu.async_copy(
                table_hbm.at[pl.ds(v[l], 1), :],
                nodes_out.at[pl.ds(base + g * 16 + l, 1), :],
                sem,
            )
        return ()

    lax.fori_loop(0, B_PER_W // 16, issue, ())

    def drain(j, _):
        pltpu.make_async_copy(
            table_hbm.at[pl.ds(0, 1), :],
            nodes_out.at[pl.ds(base, 1), :],
            sem,
        ).wait()
        return ()

    lax.fori_loop(0, B_PER_W, drain, (), unroll=8)

    @pl.when(wid == 0)
    def _():
        pltpu.make_async_copy(table_hbm.at[pl.ds(0, 1), :], char_out, csem).wait()


def kernel(node_name_ids, char_id, object_embedding):
    mesh = plsc.VectorSubcoreMesh(core_axis_name="c", subcore_axis_name="s")
    f = pl.kernel(
        _gather_body,
        mesh=mesh,
        out_type=(
            jax.ShapeDtypeStruct((N_NODES, DIM), jnp.float32),
            jax.ShapeDtypeStruct((1, DIM), jnp.float32),
        ),
        scratch_types=[
            pltpu.VMEM((B_PER_W,), jnp.int32),
            pltpu.VMEM((16,), jnp.int32),
            pltpu.SemaphoreType.DMA,
            pltpu.SemaphoreType.DMA,
        ],
        compiler_params=pltpu.CompilerParams(use_tc_tiling_on_sc=True),
    )
    node_embeddings, char_embedding = f(
        node_name_ids.astype(jnp.int32),
        char_id.astype(jnp.int32),
        object_embedding,
    )
    return (node_embeddings, char_embedding)


# speedup vs baseline: 3.7602x; 3.7602x over previous
import functools

import jax
import jax.numpy as jnp
from jax import lax
from jax.experimental import pallas as pl
from jax.experimental.pallas import tpu as pltpu
from jax.experimental.pallas import tpu_sc as plsc

NUM_EMB = 100000
N_NODES = 16384
DIM = 32
NUM_CORES = 2
NUM_SUBCORES = 16
NUM_WORKERS = NUM_CORES * NUM_SUBCORES
B_PER_W = N_NODES // NUM_WORKERS
SPLIT = 50000


def _half_body(lo, hi, merge,
               ids_hbm, cid_hbm, table_hbm, prev_nodes, prev_char,
               nodes_out, char_out,
               idx_v, cidx_v, rows_v, crow_v, sem, csem, osem):
    wid = lax.axis_index("s") * NUM_CORES + lax.axis_index("c")
    base = wid * B_PER_W

    pltpu.sync_copy(ids_hbm.at[pl.ds(base, B_PER_W)], idx_v)
    if merge:
        pltpu.sync_copy(prev_nodes.at[pl.ds(base, B_PER_W)], rows_v)

    @pl.when(wid == 0)
    def _():
        pltpu.sync_copy(cid_hbm, cidx_v.at[pl.ds(0, 1)])
        if merge:
            pltpu.sync_copy(prev_char, crow_v)
        cv = cidx_v[...]

        @pl.when(jnp.logical_and(cv[0] >= lo, cv[0] < hi))
        def _():
            pltpu.async_copy(
                table_hbm.at[pl.ds(cv[0] - lo, 1), :], crow_v, csem
            )
            pltpu.make_async_copy(
                table_hbm.at[pl.ds(0, 1), :], crow_v, csem
            ).wait()

        pltpu.sync_copy(crow_v, char_out)

    def issue(g, cnt):
        v = idx_v[pl.ds(g * 16, 16)]
        for l in range(16):
            owned = jnp.logical_and(v[l] >= lo, v[l] < hi)

            @pl.when(owned)
            def _():
                pltpu.async_copy(
                    table_hbm.at[pl.ds(v[l] - lo, 1), :],
                    rows_v.at[pl.ds(g * 16 + l, 1), :],
                    sem,
                )

            cnt = cnt + owned.astype(jnp.int32)
        return cnt

    count = lax.fori_loop(0, B_PER_W // 16, issue, jnp.int32(0))

    def drain(j, _):
        pltpu.make_async_copy(
            table_hbm.at[pl.ds(0, 1), :],
            rows_v.at[pl.ds(0, 1), :],
            sem,
        ).wait()
        return ()

    lax.fori_loop(0, count, drain, ())

    pltpu.async_copy(rows_v, nodes_out.at[pl.ds(base, B_PER_W)], osem).wait()


def _make_half(lo, hi, merge):
    n_extra = 2 if merge else 0
    body = functools.partial(_half_body, lo, hi, merge)
    if not merge:
        def body(ids, cid, table, nodes_out, char_out, *scratch):
            return _half_body(lo, hi, False, ids, cid, table, None, None,
                              nodes_out, char_out, *scratch)
    mesh = plsc.VectorSubcoreMesh(core_axis_name="c", subcore_axis_name="s")
    return pl.kernel(
        body,
        mesh=mesh,
        out_type=(
            jax.ShapeDtypeStruct((N_NODES, DIM), jnp.float32),
            jax.ShapeDtypeStruct((1, DIM), jnp.float32),
        ),
        scratch_types=[
            pltpu.VMEM((B_PER_W,), jnp.int32),
            pltpu.VMEM((16,), jnp.int32),
            pltpu.VMEM((B_PER_W, DIM), jnp.float32),
            pltpu.VMEM((1, DIM), jnp.float32),
            pltpu.SemaphoreType.DMA,
            pltpu.SemaphoreType.DMA,
            pltpu.SemaphoreType.DMA,
        ],
        compiler_params=pltpu.CompilerParams(use_tc_tiling_on_sc=True),
    )


def kernel(node_name_ids, char_id, object_embedding):
    ids = node_name_ids.astype(jnp.int32)
    cid = char_id.astype(jnp.int32)
    t0 = object_embedding[:SPLIT]
    t1 = object_embedding[SPLIT:]
    nodes0, char0 = _make_half(0, SPLIT, False)(ids, cid, t0)
    nodes, char = _make_half(SPLIT, NUM_EMB, True)(
        ids, cid, t1, nodes0, char0
    )
    return (nodes, char)
